# baseline (speedup 1.0000x reference)
import functools

import jax
import jax.numpy as jnp
from jax.experimental import pallas as pl
from jax.experimental.pallas import tpu as pltpu

NO_ELECTRON_WEIGHT = 0.1
SALIENCE_ALPHA = 0.25
SALIENCE_GAMMA = 2.0

B = 4
P = 4096
N = 512
CHUNK = 1024
NC = P // CHUNK

MASK_ELEMS = float(B * P * N)
SAL_ELEMS = float(B * 65536)
DICE_SLOTS = float(B * N)
NQ = 2048.0


def _bce(logits, targets):
    return (jnp.maximum(logits, 0.0) - logits * targets
            + jnp.log1p(jnp.exp(-jnp.abs(logits))))


def _loss_body(pred_ref, lab_ref, mask_ref, true_ref, pos_ref, post_ref,
               sal_ref, salt_ref, out_ref, acc_ref, spt_ref, sp_ref, st_ref):
    b = pl.program_id(0)
    c = pl.program_id(1)

    @pl.when(jnp.logical_and(b == 0, c == 0))
    def _init():
        acc_ref[0] = 0.0

    x = mask_ref[...]
    t = true_ref[...]
    e = jnp.exp(-jnp.abs(x))
    bce = jnp.maximum(x, 0.0) - x * t + jnp.log1p(e)
    inv1pe = 1.0 / (1.0 + e)
    probs = jnp.where(x >= 0.0, inv1pe, e * inv1pe)

    acc_ref[0] += jnp.sum(bce) * (1.0 / MASK_ELEMS)

    pt = jnp.sum(probs * t, axis=1)
    sp = jnp.sum(probs, axis=1)
    st = jnp.sum(t, axis=1)

    @pl.when(c == 0)
    def _dice_init():
        spt_ref[...] = pt
        sp_ref[...] = sp
        st_ref[...] = st

    @pl.when(c > 0)
    def _dice_acc():
        spt_ref[...] += pt
        sp_ref[...] += sp
        st_ref[...] += st

    @pl.when(c == NC - 1)
    def _dice_done():
        dice = 1.0 - (2.0 * spt_ref[...] + 1.0) / (sp_ref[...] + st_ref[...] + 1.0)
        acc_ref[0] += jnp.sum(dice) * (1.0 / DICE_SLOTS)

    @pl.when(c == 0)
    def _salience():
        s = sal_ref[...]
        st_ = salt_ref[...]
        es = jnp.exp(-jnp.abs(s))
        inv = 1.0 / (1.0 + es)
        p = jnp.where(s >= 0.0, inv, es * inv)
        ce = jnp.maximum(s, 0.0) - s * st_ + jnp.log1p(es)
        p_t = p * st_ + (1.0 - p) * (1.0 - st_)
        om = 1.0 - p_t
        alpha_t = SALIENCE_ALPHA * st_ + (1.0 - SALIENCE_ALPHA) * (1.0 - st_)
        acc_ref[0] += jnp.sum(alpha_t * ce * om * om) * (1.0 / SAL_ELEMS)

    @pl.when(jnp.logical_and(b == 0, c == 0))
    def _small():
        lab = lab_ref[...].astype(jnp.float32)
        w = jnp.where(lab == 1.0, 1.0, NO_ELECTRON_WEIGHT)
        per_q = _bce(pred_ref[...], lab)
        acc_ref[0] += jnp.sum(w * per_q) / jnp.sum(w)

        d = pos_ref[...] - post_ref[...]
        a = jnp.abs(d)
        h = jnp.where(a < 1.0, 0.5 * d * d, a - 0.5)
        acc_ref[0] += jnp.sum(h) * (1.0 / NQ)

    @pl.when(jnp.logical_and(b == B - 1, c == NC - 1))
    def _emit():
        out_ref[...] = jnp.broadcast_to(acc_ref[0], (1, 1))


@functools.partial(jax.jit, static_argnames=("interpret",))
def kernel(pred_logits, labels, mask_logits, true_masks, pred_positions,
           true_positions, salience_logits, salience_targets, interpret=False):
    pred2 = pred_logits.reshape(16, 128)
    lab2 = labels.reshape(16, 128)
    posp = pred_positions.reshape(32, 128)
    post = true_positions.reshape(32, 128)
    sal3 = salience_logits.reshape(B, 512, 128)
    salt3 = salience_targets.reshape(B, 512, 128)

    grid = (B, NC)
    out = pl.pallas_call(
        _loss_body,
        grid=grid,
        in_specs=[
            pl.BlockSpec((16, 128), lambda b, c: (0, 0)),
            pl.BlockSpec((16, 128), lambda b, c: (0, 0)),
            pl.BlockSpec((1, CHUNK, N), lambda b, c: (b, c, 0)),
            pl.BlockSpec((1, CHUNK, N), lambda b, c: (b, c, 0)),
            pl.BlockSpec((32, 128), lambda b, c: (0, 0)),
            pl.BlockSpec((32, 128), lambda b, c: (0, 0)),
            pl.BlockSpec((1, 512, 128), lambda b, c: (b, 0, 0)),
            pl.BlockSpec((1, 512, 128), lambda b, c: (b, 0, 0)),
        ],
        out_specs=pl.BlockSpec((1, 1), lambda b, c: (0, 0)),
        out_shape=jax.ShapeDtypeStruct((1, 1), jnp.float32),
        scratch_shapes=[
            pltpu.SMEM((1,), jnp.float32),
            pltpu.VMEM((1, N), jnp.float32),
            pltpu.VMEM((1, N), jnp.float32),
            pltpu.VMEM((1, N), jnp.float32),
        ],
        compiler_params=pltpu.CompilerParams(
            dimension_semantics=("arbitrary", "arbitrary"),
        ),
        interpret=interpret,
    )(pred2, lab2, mask_logits, true_masks, posp, post, sal3, salt3)
    return out.reshape(())
